# baseline (speedup 1.0000x reference)
import math

import jax
import jax.numpy as jnp
from jax.experimental import pallas as pl
from jax.experimental.pallas import tpu as pltpu

_VOCAB = 100000
_SMOOTHING = 0.1
_CONF = 1.0 - _SMOOTHING
_EPS = _SMOOTHING / (_VOCAB - 2)
_C1 = (_VOCAB - 2) * _EPS * math.log(_EPS) + _CONF * math.log(_CONF)

_N = 1024
_BC = 2048
_NBLK = (_VOCAB + _BC - 1) // _BC


def _kl_body(x_ref, t_ref, o_ref, sacc, gacc, x0):
    j = pl.program_id(0)

    @pl.when(j == 0)
    def _init():
        sacc[...] = jnp.zeros_like(sacc)
        gacc[...] = jnp.zeros_like(gacc)
        x0[...] = x_ref[:, 0:1]

    xb = x_ref[...]

    @pl.when(j < _NBLK - 1)
    def _full():
        sacc[...] += jnp.sum(xb, axis=1, keepdims=True)

    @pl.when(j == _NBLK - 1)
    def _tail():
        valid = jax.lax.broadcasted_iota(jnp.int32, (_N, _BC), 1) < (
            _VOCAB - (_NBLK - 1) * _BC
        )
        sacc[...] += jnp.sum(jnp.where(valid, xb, 0.0), axis=1, keepdims=True)

    tgt = t_ref[...]
    idx = tgt - j * _BC
    inb = (idx >= 0) & (idx < _BC)
    idxc = jnp.clip(idx, 0, _BC - 1)
    qt = idxc // 128
    rm = idxc % 128
    g = jnp.zeros((_N, 1), jnp.float32)
    for k in range(_BC // 128):
        gk = jnp.take_along_axis(xb[:, k * 128 : (k + 1) * 128], rm, axis=1)
        g = jnp.where(qt == k, gk, g)
    gacc[...] = jnp.where(inb, g, gacc[...])

    @pl.when(j == _NBLK - 1)
    def _finish():
        m = t_ref[...] != 0
        per_row = _C1 - _EPS * (sacc[...] - x0[...] - gacc[...]) - _CONF * gacc[...]
        o_ref[0, 0] = jnp.sum(jnp.where(m, per_row, 0.0))


def kernel(x, target):
    n, v = x.shape
    tgt2 = target.astype(jnp.int32).reshape(n, 1)
    out = pl.pallas_call(
        _kl_body,
        grid=(_NBLK,),
        in_specs=[
            pl.BlockSpec((n, _BC), lambda j: (0, j)),
            pl.BlockSpec((n, 1), lambda j: (0, 0)),
        ],
        out_specs=pl.BlockSpec(memory_space=pltpu.SMEM),
        out_shape=jax.ShapeDtypeStruct((1, 1), jnp.float32),
        scratch_shapes=[
            pltpu.VMEM((n, 1), jnp.float32),
            pltpu.VMEM((n, 1), jnp.float32),
            pltpu.VMEM((n, 1), jnp.float32),
        ],
    )(x, tgt2)
    return out[0, 0]


# speedup vs baseline: 1.4366x; 1.4366x over previous
import math

import jax
import jax.numpy as jnp
from jax.experimental import pallas as pl
from jax.experimental.pallas import tpu as pltpu

_VOCAB = 100000
_SMOOTHING = 0.1
_CONF = 1.0 - _SMOOTHING
_EPS = _SMOOTHING / (_VOCAB - 2)
_C1 = (_VOCAB - 2) * _EPS * math.log(_EPS) + _CONF * math.log(_CONF)

_N = 1024
_BC = 2048
_NBLK = (_VOCAB + _BC - 1) // _BC


def _kl_body(x_ref, t_ref, o_ref, sacc, gacc, x0):
    j = pl.program_id(0)

    @pl.when(j == 0)
    def _init():
        sacc[...] = jnp.zeros_like(sacc)
        gacc[...] = jnp.zeros_like(gacc)
        x0[...] = x_ref[:, 0:1]

    xb = x_ref[...]
    idx = t_ref[...] - j * _BC
    lane = jax.lax.broadcasted_iota(jnp.int32, (_N, _BC), 1)
    hit = lane == idx
    gacc[...] += jnp.sum(jnp.where(hit, xb, 0.0), axis=1, keepdims=True)

    @pl.when(j < _NBLK - 1)
    def _full():
        sacc[...] += jnp.sum(xb, axis=1, keepdims=True)

    @pl.when(j == _NBLK - 1)
    def _tail():
        valid = lane < (_VOCAB - (_NBLK - 1) * _BC)
        sacc[...] += jnp.sum(jnp.where(valid, xb, 0.0), axis=1, keepdims=True)

        m = t_ref[...] != 0
        wsum = _EPS * (sacc[...] - x0[...]) + (_CONF - _EPS) * gacc[...]
        o_ref[0, 0] = jnp.sum(jnp.where(m, _C1 - wsum, 0.0))


def kernel(x, target):
    n, v = x.shape
    tgt2 = target.astype(jnp.int32).reshape(n, 1)
    out = pl.pallas_call(
        _kl_body,
        grid=(_NBLK,),
        in_specs=[
            pl.BlockSpec((n, _BC), lambda j: (0, j)),
            pl.BlockSpec((n, 1), lambda j: (0, 0)),
        ],
        out_specs=pl.BlockSpec(memory_space=pltpu.SMEM),
        out_shape=jax.ShapeDtypeStruct((1, 1), jnp.float32),
        scratch_shapes=[
            pltpu.VMEM((n, 1), jnp.float32),
            pltpu.VMEM((n, 1), jnp.float32),
            pltpu.VMEM((n, 1), jnp.float32),
        ],
    )(x, tgt2)
    return out[0, 0]


# speedup vs baseline: 1.4469x; 1.0072x over previous
import math

import jax
import jax.numpy as jnp
from jax.experimental import pallas as pl
from jax.experimental.pallas import tpu as pltpu

_VOCAB = 100000
_SMOOTHING = 0.1
_CONF = 1.0 - _SMOOTHING
_EPS = _SMOOTHING / (_VOCAB - 2)
_C1 = (_VOCAB - 2) * _EPS * math.log(_EPS) + _CONF * math.log(_CONF)

_N = 1024
_BC = 4096
_NBLK = (_VOCAB + _BC - 1) // _BC


def _kl_body(x_ref, t_ref, o_ref, sacc, gacc, x0):
    j = pl.program_id(0)

    @pl.when(j == 0)
    def _init():
        sacc[...] = jnp.zeros_like(sacc)
        gacc[...] = jnp.zeros_like(gacc)
        x0[...] = x_ref[:, 0:1]

    xb = x_ref[...]
    idx = t_ref[...] - j * _BC
    lane = jax.lax.broadcasted_iota(jnp.int32, (_N, _BC), 1)
    hit = lane == idx
    gacc[...] += jnp.sum(jnp.where(hit, xb, 0.0), axis=1, keepdims=True)

    @pl.when(j < _NBLK - 1)
    def _full():
        sacc[...] += jnp.sum(xb, axis=1, keepdims=True)

    @pl.when(j == _NBLK - 1)
    def _tail():
        valid = lane < (_VOCAB - (_NBLK - 1) * _BC)
        sacc[...] += jnp.sum(jnp.where(valid, xb, 0.0), axis=1, keepdims=True)

        m = t_ref[...] != 0
        wsum = _EPS * (sacc[...] - x0[...]) + (_CONF - _EPS) * gacc[...]
        o_ref[0, 0] = jnp.sum(jnp.where(m, _C1 - wsum, 0.0))


def kernel(x, target):
    n, v = x.shape
    tgt2 = target.astype(jnp.int32).reshape(n, 1)
    out = pl.pallas_call(
        _kl_body,
        grid=(_NBLK,),
        in_specs=[
            pl.BlockSpec((n, _BC), lambda j: (0, j)),
            pl.BlockSpec((n, 1), lambda j: (0, 0)),
        ],
        out_specs=pl.BlockSpec(memory_space=pltpu.SMEM),
        out_shape=jax.ShapeDtypeStruct((1, 1), jnp.float32),
        scratch_shapes=[
            pltpu.VMEM((n, 1), jnp.float32),
            pltpu.VMEM((n, 1), jnp.float32),
            pltpu.VMEM((n, 1), jnp.float32),
        ],
    )(x, tgt2)
    return out[0, 0]
